# baseline (speedup 1.0000x reference)
import functools

import jax
import jax.numpy as jnp
from jax import lax
from jax.experimental import pallas as pl
from jax.experimental.pallas import tpu as pltpu
from jax.experimental.pallas import tpu_sc as plsc

N = 10000
NPAD = 10112
E = 320000
CHUNK = 128
NROWS = 2560
EPAD = NROWS * CHUNK
NC, NS = 2, 16
NW = NC * NS
CPW = NROWS // NW
NBUF = 4
RPT = NPAD // NS
SPLIT64 = 230
SPLIT128 = 108
RBLK = 2528
GRID = NPAD // RBLK
G = 128
OUT = 10


def _make_sc_pass(F, gather, chunk, nbuf, split0=None, dtype=jnp.float32):
    cpp = EPAD // chunk // NS
    cpw0 = (cpp // 2 if split0 is None else split0)
    cpw1 = cpp - cpw0
    assert cpw0 % nbuf == 0 and cpw1 % nbuf == 0
    mesh = plsc.VectorSubcoreMesh(core_axis_name="c", subcore_axis_name="s",
                                  num_cores=NC, num_subcores=NS)

    @functools.partial(
        pl.kernel,
        out_type=jax.ShapeDtypeStruct((NC, NPAD, F), dtype),
        mesh=mesh,
        compiler_params=pltpu.CompilerParams(use_tc_tiling_on_sc=False),
        scratch_types=[
            pltpu.VMEM((2, nbuf, chunk), jnp.int32),
            pltpu.VMEM((2, nbuf, chunk), jnp.int32),
            [pltpu.VMEM((chunk, F), dtype) for _ in range(nbuf)],
            pltpu.VMEM_SHARED((NPAD, F), dtype),
            [pltpu.SemaphoreType.DMA for _ in range(nbuf)],
            [pltpu.SemaphoreType.DMA for _ in range(nbuf)],
            pltpu.SemaphoreType.DMA,
        ],
    )
    def sc_pass(init_hbm, src_hbm, row_hbm, col_hbm, out_hbm,
                idx_r, idx_c, rows, acc, gsem, ssem, isem):
        cid = lax.axis_index("c")
        sid = lax.axis_index("s")
        sl = pl.ds(sid * RPT, RPT)
        pltpu.sync_copy(init_hbm.at[sl], acc.at[sl])
        base = sid * cpp + jnp.where(cid == 0, 0, cpw0)
        nrnd = jnp.where(cid == 0, cpw0 // nbuf, cpw1 // nbuf)
        pltpu.sync_copy(col_hbm.at[pl.ds(base, nbuf)], idx_c.at[0])
        if gather:
            pltpu.sync_copy(row_hbm.at[pl.ds(base, nbuf)], idx_r.at[0])
        else:
            for b in range(nbuf):
                pltpu.sync_copy(init_hbm.at[pl.ds(0, chunk)], rows[b])
        plsc.subcore_barrier()

        def round_(r, carry):
            rp = lax.rem(r, 2)
            rn = lax.rem(r + 1, 2)

            @pl.when(r + 1 < nrnd)
            def _():
                nxt = pl.ds(base + (r + 1) * nbuf, nbuf)
                pltpu.async_copy(col_hbm.at[nxt], idx_c.at[rn], isem)
                if gather:
                    pltpu.async_copy(row_hbm.at[nxt], idx_r.at[rn], isem)

            if gather:
                for b in range(nbuf):
                    @pl.when(r > 0)
                    def _(b=b):
                        pltpu.make_async_copy(
                            rows[b], acc.at[idx_c.at[0, b]], ssem[b]).wait()
                    pltpu.async_copy(
                        src_hbm.at[idx_r.at[rp, b]], rows[b], gsem[b])
                for b in range(nbuf):
                    pltpu.make_async_copy(
                        src_hbm.at[idx_r.at[0, b]], rows[b], gsem[b]).wait()
                    pltpu.async_copy(
                        rows[b], acc.at[idx_c.at[rp, b]], ssem[b], add=True)
            else:
                for b in range(nbuf):
                    @pl.when(r > 0)
                    def _(b=b):
                        pltpu.make_async_copy(
                            rows[b], acc.at[idx_c.at[0, b]], ssem[b]).wait()
                    pltpu.async_copy(
                        rows[b], acc.at[idx_c.at[rp, b]], ssem[b], add=True)

            @pl.when(r + 1 < nrnd)
            def _():
                pltpu.make_async_copy(
                    col_hbm.at[pl.ds(0, nbuf)], idx_c.at[0], isem).wait()
                if gather:
                    pltpu.make_async_copy(
                        row_hbm.at[pl.ds(0, nbuf)], idx_r.at[0], isem).wait()
            return carry

        lax.fori_loop(0, nrnd, round_, 0)
        for b in range(nbuf):
            pltpu.make_async_copy(rows[b], acc.at[idx_c.at[0, b]],
                                  ssem[b]).wait()
        plsc.subcore_barrier()
        pltpu.sync_copy(acc.at[sl], out_hbm.at[cid, sl])

    return sc_pass


_SC_CACHE = {}


def _sc_pass(F, gather, chunk, nbuf, split0, dtype, *args):
    key = (F, gather, chunk, nbuf, split0, dtype)
    if key not in _SC_CACHE:
        _SC_CACHE[key] = _make_sc_pass(F, gather, chunk, nbuf, split0, dtype)
    return _SC_CACHE[key](*args)


def _tc_prep(degp, x, W1):
    def body(degp_ref, x_ref, w_ref, dis_ref, v_ref):
        deg = degp_ref[0, :, 0:1] + degp_ref[1, :, 0:1]
        dis = lax.rsqrt(deg)
        dis_ref[...] = jnp.broadcast_to(dis, (RBLK, 8))
        xw = jnp.dot(x_ref[...], w_ref[...], preferred_element_type=jnp.float32)
        v_ref[...] = (dis * xw).astype(v_ref.dtype)

    return pl.pallas_call(
        body,
        grid=(GRID,),
        in_specs=[pl.BlockSpec((2, RBLK, 16), lambda i: (0, i, 0)),
                  pl.BlockSpec((RBLK, 128), lambda i: (i, 0)),
                  pl.BlockSpec((128, 160), lambda i: (0, 0))],
        out_specs=[pl.BlockSpec((RBLK, 8), lambda i: (i, 0)),
                   pl.BlockSpec((RBLK, 160), lambda i: (i, 0))],
        out_shape=[jax.ShapeDtypeStruct((NPAD, 8), jnp.float32),
                   jax.ShapeDtypeStruct((NPAD, 160), jnp.bfloat16)],
    )(degp, x, W1)


def _tc_combine(S, dis8, b, W, fin, fout):
    def body(s_ref, dis_ref, b_ref, w_ref, o_ref):
        dis = dis_ref[:, 0:1]
        s = (s_ref[0].astype(jnp.float32) + s_ref[1].astype(jnp.float32))
        h = dis * s + b_ref[...]
        o_ref[...] = (dis * jnp.dot(h, w_ref[...],
                                    preferred_element_type=jnp.float32)
                      ).astype(o_ref.dtype)

    return pl.pallas_call(
        body,
        grid=(GRID,),
        in_specs=[pl.BlockSpec((2, RBLK, fin), lambda i: (0, i, 0)),
                  pl.BlockSpec((RBLK, 8), lambda i: (i, 0)),
                  pl.BlockSpec((1, fin), lambda i: (0, 0)),
                  pl.BlockSpec((fin, fout), lambda i: (0, 0))],
        out_specs=pl.BlockSpec((RBLK, fout), lambda i: (i, 0)),
        out_shape=jax.ShapeDtypeStruct((NPAD, fout), jnp.bfloat16),
    )(S, dis8, b, W)


def _tc_final(S, dis8, b3, batch2d, fcW, fcb):
    def body(s_ref, dis_ref, b_ref, batch_ref, fcw_ref, fcb_ref,
             out_ref, sums, cnts):
        i = pl.program_id(0)

        @pl.when(i == 0)
        def _():
            sums[...] = jnp.zeros_like(sums)
            cnts[...] = jnp.zeros_like(cnts)

        dis = dis_ref[:, 0:1]
        s = (s_ref[0].astype(jnp.float32) + s_ref[1].astype(jnp.float32))
        h = dis * s + b_ref[...]
        oh = (batch_ref[...] ==
              lax.broadcasted_iota(jnp.int32, (RBLK, G), 1)
              ).astype(jnp.float32)
        dnum = (((0,), (0,)), ((), ()))
        sums[...] += lax.dot_general(oh, h, dnum,
                                     preferred_element_type=jnp.float32)
        cnts[...] += lax.dot_general(oh, jnp.ones((RBLK, G), jnp.float32),
                                     dnum, preferred_element_type=jnp.float32)

        @pl.when(i == GRID - 1)
        def _():
            pooled = sums[...] / jnp.maximum(cnts[...], 1.0)
            logits = jnp.dot(pooled, fcw_ref[...],
                             preferred_element_type=jnp.float32) + fcb_ref[...]
            m = jnp.max(logits, axis=1, keepdims=True)
            lse = jnp.log(jnp.sum(jnp.exp(logits - m), axis=1,
                                  keepdims=True)) + m
            out_ref[...] = logits - lse

    return pl.pallas_call(
        body,
        grid=(GRID,),
        in_specs=[pl.BlockSpec((2, RBLK, G), lambda i: (0, i, 0)),
                  pl.BlockSpec((RBLK, 8), lambda i: (i, 0)),
                  pl.BlockSpec((1, G), lambda i: (0, 0)),
                  pl.BlockSpec((RBLK, 1), lambda i: (i, 0)),
                  pl.BlockSpec((G, OUT), lambda i: (0, 0)),
                  pl.BlockSpec((1, OUT), lambda i: (0, 0))],
        out_specs=pl.BlockSpec((G, OUT), lambda i: (0, 0)),
        out_shape=jax.ShapeDtypeStruct((G, OUT), jnp.float32),
        scratch_shapes=[pltpu.VMEM((G, G), jnp.float32),
                        pltpu.VMEM((G, G), jnp.float32)],
    )(S, dis8, b3, batch2d, fcW, fcb)


def kernel(x, edge_index, batch, W1, b1, W2, b2, W3, b3, fcW, fcb):
    xp = jnp.concatenate([x, jnp.zeros((NPAD - N, x.shape[1]), jnp.float32)])
    rowflat = jnp.concatenate([edge_index[0], jnp.zeros((EPAD - E,), jnp.int32)])
    colflat = jnp.concatenate([edge_index[1], jnp.full((EPAD - E,), N, jnp.int32)])
    row128 = rowflat.reshape(-1, 128)
    col128 = colflat.reshape(-1, 128)
    row64 = rowflat.reshape(-1, 64)
    col64 = colflat.reshape(-1, 64)
    batch2d = jnp.concatenate(
        [batch, jnp.full((NPAD - N,), G, jnp.int32)]).reshape(NPAD, 1)
    ones16 = jnp.ones((NPAD, 16), jnp.float32)

    BF = jnp.bfloat16
    F32 = jnp.float32
    degp = _sc_pass(16, False, 128, 4, None, F32, ones16, ones16,
                    row128, col128)
    dis8, v1 = _tc_prep(degp, xp, W1)
    S1 = _sc_pass(160, True, 128, 4, SPLIT128, BF, v1, v1, row128, col128)
    v2 = _tc_combine(S1, dis8, b1.reshape(1, -1), W2, 160, 160)
    S2 = _sc_pass(160, True, 128, 4, SPLIT128, BF, v2, v2, row128, col128)
    v3 = _tc_combine(S2, dis8, b2.reshape(1, -1), W3, 160, 128)
    S3 = _sc_pass(128, True, 128, 4, SPLIT128, BF, v3, v3, row128, col128)
    return _tc_final(S3, dis8, b3.reshape(1, -1), batch2d,
                     fcW, fcb.reshape(1, -1))


# speedup vs baseline: 1.0606x; 1.0606x over previous
import functools

import jax
import jax.numpy as jnp
from jax import lax
from jax.experimental import pallas as pl
from jax.experimental.pallas import tpu as pltpu
from jax.experimental.pallas import tpu_sc as plsc

N = 10000
NPAD = 10112
E = 320000
CHUNK = 128
NROWS = 2560
EPAD = NROWS * CHUNK
NC, NS = 2, 16
NW = NC * NS
CPW = NROWS // NW
NBUF = 4
RPT = NPAD // NS
SPLIT64 = 230
SPLIT128 = 116
RBLK = 2528
GRID = NPAD // RBLK
G = 128
OUT = 10


def _make_sc_pass(F, gather, chunk, nbuf, split0=None, dtype=jnp.float32):
    cpp = EPAD // chunk // NS
    cpw0 = (cpp // 2 if split0 is None else split0)
    cpw1 = cpp - cpw0
    assert cpw0 % nbuf == 0 and cpw1 % nbuf == 0
    mesh = plsc.VectorSubcoreMesh(core_axis_name="c", subcore_axis_name="s",
                                  num_cores=NC, num_subcores=NS)

    @functools.partial(
        pl.kernel,
        out_type=jax.ShapeDtypeStruct((NC, NPAD, F), dtype),
        mesh=mesh,
        compiler_params=pltpu.CompilerParams(use_tc_tiling_on_sc=False),
        scratch_types=[
            pltpu.VMEM((2, nbuf, chunk), jnp.int32),
            pltpu.VMEM((2, nbuf, chunk), jnp.int32),
            [pltpu.VMEM((chunk, F), dtype) for _ in range(nbuf)],
            pltpu.VMEM_SHARED((NPAD, F), dtype),
            [pltpu.SemaphoreType.DMA for _ in range(nbuf)],
            [pltpu.SemaphoreType.DMA for _ in range(nbuf)],
            pltpu.SemaphoreType.DMA,
        ],
    )
    def sc_pass(init_hbm, src_hbm, row_hbm, col_hbm, out_hbm,
                idx_r, idx_c, rows, acc, gsem, ssem, isem):
        cid = lax.axis_index("c")
        sid = lax.axis_index("s")
        sl = pl.ds(sid * RPT, RPT)
        pltpu.sync_copy(init_hbm.at[sl], acc.at[sl])
        base = sid * cpp + jnp.where(cid == 0, 0, cpw0)
        nrnd = jnp.where(cid == 0, cpw0 // nbuf, cpw1 // nbuf)
        pltpu.sync_copy(col_hbm.at[pl.ds(base, nbuf)], idx_c.at[0])
        if gather:
            pltpu.sync_copy(row_hbm.at[pl.ds(base, nbuf)], idx_r.at[0])
        else:
            for b in range(nbuf):
                pltpu.sync_copy(init_hbm.at[pl.ds(0, chunk)], rows[b])
        plsc.subcore_barrier()

        def round_(r, carry):
            rp = lax.rem(r, 2)
            rn = lax.rem(r + 1, 2)

            @pl.when(r + 1 < nrnd)
            def _():
                nxt = pl.ds(base + (r + 1) * nbuf, nbuf)
                pltpu.async_copy(col_hbm.at[nxt], idx_c.at[rn], isem)
                if gather:
                    pltpu.async_copy(row_hbm.at[nxt], idx_r.at[rn], isem)

            if gather:
                for b in range(nbuf):
                    @pl.when(r > 0)
                    def _(b=b):
                        pltpu.make_async_copy(
                            rows[b], acc.at[idx_c.at[0, b]], ssem[b]).wait()
                    pltpu.async_copy(
                        src_hbm.at[idx_r.at[rp, b]], rows[b], gsem[b])
                for b in range(nbuf):
                    pltpu.make_async_copy(
                        src_hbm.at[idx_r.at[0, b]], rows[b], gsem[b]).wait()
                    pltpu.async_copy(
                        rows[b], acc.at[idx_c.at[rp, b]], ssem[b], add=True)
            else:
                for b in range(nbuf):
                    @pl.when(r > 0)
                    def _(b=b):
                        pltpu.make_async_copy(
                            rows[b], acc.at[idx_c.at[0, b]], ssem[b]).wait()
                    pltpu.async_copy(
                        rows[b], acc.at[idx_c.at[rp, b]], ssem[b], add=True)

            @pl.when(r + 1 < nrnd)
            def _():
                pltpu.make_async_copy(
                    col_hbm.at[pl.ds(0, nbuf)], idx_c.at[0], isem).wait()
                if gather:
                    pltpu.make_async_copy(
                        row_hbm.at[pl.ds(0, nbuf)], idx_r.at[0], isem).wait()
            return carry

        lax.fori_loop(0, nrnd, round_, 0)
        for b in range(nbuf):
            pltpu.make_async_copy(rows[b], acc.at[idx_c.at[0, b]],
                                  ssem[b]).wait()
        plsc.subcore_barrier()
        pltpu.sync_copy(acc.at[sl], out_hbm.at[cid, sl])

    return sc_pass


_SC_CACHE = {}


def _sc_pass(F, gather, chunk, nbuf, split0, dtype, *args):
    key = (F, gather, chunk, nbuf, split0, dtype)
    if key not in _SC_CACHE:
        _SC_CACHE[key] = _make_sc_pass(F, gather, chunk, nbuf, split0, dtype)
    return _SC_CACHE[key](*args)


def _tc_prep(degp, x, W1):
    def body(degp_ref, x_ref, w_ref, dis_ref, v_ref):
        deg = degp_ref[0, :, 0:1] + degp_ref[1, :, 0:1]
        dis = lax.rsqrt(deg)
        dis_ref[...] = jnp.broadcast_to(dis, (RBLK, 8))
        xw = jnp.dot(x_ref[...], w_ref[...], preferred_element_type=jnp.float32)
        v_ref[...] = (dis * xw).astype(v_ref.dtype)

    return pl.pallas_call(
        body,
        grid=(GRID,),
        in_specs=[pl.BlockSpec((2, RBLK, 16), lambda i: (0, i, 0)),
                  pl.BlockSpec((RBLK, 128), lambda i: (i, 0)),
                  pl.BlockSpec((128, 160), lambda i: (0, 0))],
        out_specs=[pl.BlockSpec((RBLK, 8), lambda i: (i, 0)),
                   pl.BlockSpec((RBLK, 160), lambda i: (i, 0))],
        out_shape=[jax.ShapeDtypeStruct((NPAD, 8), jnp.float32),
                   jax.ShapeDtypeStruct((NPAD, 160), jnp.bfloat16)],
    )(degp, x, W1)


def _tc_combine(S, dis8, b, W, fin, fout):
    def body(s_ref, dis_ref, b_ref, w_ref, o_ref):
        dis = dis_ref[:, 0:1]
        s = (s_ref[0].astype(jnp.float32) + s_ref[1].astype(jnp.float32))
        h = dis * s + b_ref[...]
        o_ref[...] = (dis * jnp.dot(h, w_ref[...],
                                    preferred_element_type=jnp.float32)
                      ).astype(o_ref.dtype)

    return pl.pallas_call(
        body,
        grid=(GRID,),
        in_specs=[pl.BlockSpec((2, RBLK, fin), lambda i: (0, i, 0)),
                  pl.BlockSpec((RBLK, 8), lambda i: (i, 0)),
                  pl.BlockSpec((1, fin), lambda i: (0, 0)),
                  pl.BlockSpec((fin, fout), lambda i: (0, 0))],
        out_specs=pl.BlockSpec((RBLK, fout), lambda i: (i, 0)),
        out_shape=jax.ShapeDtypeStruct((NPAD, fout), jnp.bfloat16),
    )(S, dis8, b, W)


def _tc_final(S, dis8, b3, batch2d, fcW, fcb):
    def body(s_ref, dis_ref, b_ref, batch_ref, fcw_ref, fcb_ref,
             out_ref, sums, cnts):
        i = pl.program_id(0)

        @pl.when(i == 0)
        def _():
            sums[...] = jnp.zeros_like(sums)
            cnts[...] = jnp.zeros_like(cnts)

        dis = dis_ref[:, 0:1]
        s = (s_ref[0].astype(jnp.float32) + s_ref[1].astype(jnp.float32))
        h = dis * s + b_ref[...]
        oh = (batch_ref[...] ==
              lax.broadcasted_iota(jnp.int32, (RBLK, G), 1)
              ).astype(jnp.float32)
        dnum = (((0,), (0,)), ((), ()))
        sums[...] += lax.dot_general(oh, h, dnum,
                                     preferred_element_type=jnp.float32)
        cnts[...] += lax.dot_general(oh, jnp.ones((RBLK, G), jnp.float32),
                                     dnum, preferred_element_type=jnp.float32)

        @pl.when(i == GRID - 1)
        def _():
            pooled = sums[...] / jnp.maximum(cnts[...], 1.0)
            logits = jnp.dot(pooled, fcw_ref[...],
                             preferred_element_type=jnp.float32) + fcb_ref[...]
            m = jnp.max(logits, axis=1, keepdims=True)
            lse = jnp.log(jnp.sum(jnp.exp(logits - m), axis=1,
                                  keepdims=True)) + m
            out_ref[...] = logits - lse

    return pl.pallas_call(
        body,
        grid=(GRID,),
        in_specs=[pl.BlockSpec((2, RBLK, G), lambda i: (0, i, 0)),
                  pl.BlockSpec((RBLK, 8), lambda i: (i, 0)),
                  pl.BlockSpec((1, G), lambda i: (0, 0)),
                  pl.BlockSpec((RBLK, 1), lambda i: (i, 0)),
                  pl.BlockSpec((G, OUT), lambda i: (0, 0)),
                  pl.BlockSpec((1, OUT), lambda i: (0, 0))],
        out_specs=pl.BlockSpec((G, OUT), lambda i: (0, 0)),
        out_shape=jax.ShapeDtypeStruct((G, OUT), jnp.float32),
        scratch_shapes=[pltpu.VMEM((G, G), jnp.float32),
                        pltpu.VMEM((G, G), jnp.float32)],
    )(S, dis8, b3, batch2d, fcW, fcb)


def kernel(x, edge_index, batch, W1, b1, W2, b2, W3, b3, fcW, fcb):
    xp = jnp.concatenate([x, jnp.zeros((NPAD - N, x.shape[1]), jnp.float32)])
    rowflat = jnp.concatenate([edge_index[0], jnp.zeros((EPAD - E,), jnp.int32)])
    colflat = jnp.concatenate([edge_index[1], jnp.full((EPAD - E,), N, jnp.int32)])
    row128 = rowflat.reshape(-1, 128)
    col128 = colflat.reshape(-1, 128)
    row64 = rowflat.reshape(-1, 64)
    col64 = colflat.reshape(-1, 64)
    batch2d = jnp.concatenate(
        [batch, jnp.full((NPAD - N,), G, jnp.int32)]).reshape(NPAD, 1)
    ones16 = jnp.ones((NPAD, 16), jnp.float32)

    BF = jnp.bfloat16
    F32 = jnp.float32
    degp = _sc_pass(16, False, 128, 4, None, F32, ones16, ones16,
                    row128, col128)
    dis8, v1 = _tc_prep(degp, xp, W1)
    S1 = _sc_pass(160, True, 128, 4, SPLIT128, BF, v1, v1, row128, col128)
    v2 = _tc_combine(S1, dis8, b1.reshape(1, -1), W2, 160, 160)
    S2 = _sc_pass(160, True, 128, 4, SPLIT128, BF, v2, v2, row128, col128)
    v3 = _tc_combine(S2, dis8, b2.reshape(1, -1), W3, 160, 128)
    S3 = _sc_pass(128, True, 128, 4, SPLIT128, BF, v3, v3, row128, col128)
    return _tc_final(S3, dis8, b3.reshape(1, -1), batch2d,
                     fcW, fcb.reshape(1, -1))


# speedup vs baseline: 1.0616x; 1.0010x over previous
import functools

import jax
import jax.numpy as jnp
from jax import lax
from jax.experimental import pallas as pl
from jax.experimental.pallas import tpu as pltpu
from jax.experimental.pallas import tpu_sc as plsc

N = 10000
NPAD = 10112
E = 320000
CHUNK = 128
NROWS = 2560
EPAD = NROWS * CHUNK
NC, NS = 2, 16
RPT = NPAD // NS
SPLIT128 = 116
RBLK = 2528
GRID = NPAD // RBLK
G = 128
OUT = 10


def _make_sc_pass(F, gather, chunk, nbuf, split0=None, dtype=jnp.float32):
    cpp = EPAD // chunk // NS
    cpw0 = (cpp // 2 if split0 is None else split0)
    cpw1 = cpp - cpw0
    assert cpw0 % nbuf == 0 and cpw1 % nbuf == 0
    mesh = plsc.VectorSubcoreMesh(core_axis_name="c", subcore_axis_name="s",
                                  num_cores=NC, num_subcores=NS)

    @functools.partial(
        pl.kernel,
        out_type=jax.ShapeDtypeStruct((NC, NPAD, F), dtype),
        mesh=mesh,
        compiler_params=pltpu.CompilerParams(use_tc_tiling_on_sc=False),
        scratch_types=[
            pltpu.VMEM((2, nbuf, chunk), jnp.int32),
            pltpu.VMEM((2, nbuf, chunk), jnp.int32),
            [pltpu.VMEM((chunk, F), dtype) for _ in range(nbuf)],
            pltpu.VMEM_SHARED((NPAD, F), dtype),
            [pltpu.SemaphoreType.DMA for _ in range(nbuf)],
            [pltpu.SemaphoreType.DMA for _ in range(nbuf)],
            pltpu.SemaphoreType.DMA,
        ],
    )
    def sc_pass(init_hbm, src_hbm, row_hbm, col_hbm, out_hbm,
                idx_r, idx_c, rows, acc, gsem, ssem, isem):
        cid = lax.axis_index("c")
        sid = lax.axis_index("s")
        sl = pl.ds(sid * RPT, RPT)
        pltpu.sync_copy(init_hbm.at[sl], acc.at[sl])
        base = sid * cpp + jnp.where(cid == 0, 0, cpw0)
        nrnd = jnp.where(cid == 0, cpw0 // nbuf, cpw1 // nbuf)
        pltpu.sync_copy(col_hbm.at[pl.ds(base, nbuf)], idx_c.at[0])
        if gather:
            pltpu.sync_copy(row_hbm.at[pl.ds(base, nbuf)], idx_r.at[0])
        else:
            for b in range(nbuf):
                pltpu.sync_copy(init_hbm.at[pl.ds(0, chunk)], rows[b])
        plsc.subcore_barrier()

        def round_(r, carry):
            rp = lax.rem(r, 2)
            rn = lax.rem(r + 1, 2)

            @pl.when(r + 1 < nrnd)
            def _():
                nxt = pl.ds(base + (r + 1) * nbuf, nbuf)
                pltpu.async_copy(col_hbm.at[nxt], idx_c.at[rn], isem)
                if gather:
                    pltpu.async_copy(row_hbm.at[nxt], idx_r.at[rn], isem)

            if gather:
                for b in range(nbuf):
                    @pl.when(r > 0)
                    def _(b=b):
                        pltpu.make_async_copy(
                            rows[b], acc.at[idx_c.at[0, b]], ssem[b]).wait()
                    pltpu.async_copy(
                        src_hbm.at[idx_r.at[rp, b]], rows[b], gsem[b])
                for b in range(nbuf):
                    pltpu.make_async_copy(
                        src_hbm.at[idx_r.at[0, b]], rows[b], gsem[b]).wait()
                    pltpu.async_copy(
                        rows[b], acc.at[idx_c.at[rp, b]], ssem[b], add=True)
            else:
                for b in range(nbuf):
                    @pl.when(r > 0)
                    def _(b=b):
                        pltpu.make_async_copy(
                            rows[b], acc.at[idx_c.at[0, b]], ssem[b]).wait()
                    pltpu.async_copy(
                        rows[b], acc.at[idx_c.at[rp, b]], ssem[b], add=True)

            @pl.when(r + 1 < nrnd)
            def _():
                pltpu.make_async_copy(
                    col_hbm.at[pl.ds(0, nbuf)], idx_c.at[0], isem).wait()
                if gather:
                    pltpu.make_async_copy(
                        row_hbm.at[pl.ds(0, nbuf)], idx_r.at[0], isem).wait()
            return carry

        lax.fori_loop(0, nrnd, round_, 0)
        for b in range(nbuf):
            pltpu.make_async_copy(rows[b], acc.at[idx_c.at[0, b]],
                                  ssem[b]).wait()
        plsc.subcore_barrier()
        pltpu.sync_copy(acc.at[sl], out_hbm.at[cid, sl])

    return sc_pass


_SC_CACHE = {}


def _sc_pass(F, gather, chunk, nbuf, split0, dtype, *args):
    key = (F, gather, chunk, nbuf, split0, dtype)
    if key not in _SC_CACHE:
        _SC_CACHE[key] = _make_sc_pass(F, gather, chunk, nbuf, split0, dtype)
    return _SC_CACHE[key](*args)


def _tc_prep(degp, x, W1):
    def body(degp_ref, x_ref, w_ref, dis_ref, v_ref):
        deg = degp_ref[0, :, 0:1] + degp_ref[1, :, 0:1]
        dis = lax.rsqrt(deg)
        dis_ref[...] = jnp.broadcast_to(dis, (RBLK, 8))
        xw = jnp.dot(x_ref[...], w_ref[...], preferred_element_type=jnp.float32)
        v_ref[...] = (dis * xw).astype(v_ref.dtype)

    return pl.pallas_call(
        body,
        grid=(GRID,),
        in_specs=[pl.BlockSpec((2, RBLK, 16), lambda i: (0, i, 0)),
                  pl.BlockSpec((RBLK, 128), lambda i: (i, 0)),
                  pl.BlockSpec((128, 160), lambda i: (0, 0))],
        out_specs=[pl.BlockSpec((RBLK, 8), lambda i: (i, 0)),
                   pl.BlockSpec((RBLK, 160), lambda i: (i, 0))],
        out_shape=[jax.ShapeDtypeStruct((NPAD, 8), jnp.float32),
                   jax.ShapeDtypeStruct((NPAD, 160), jnp.bfloat16)],
    )(degp, x, W1)


def _tc_combine(S, dis8, b, W, fin, fout):
    def body(s_ref, dis_ref, b_ref, w_ref, o_ref):
        dis = dis_ref[:, 0:1]
        s = (s_ref[0].astype(jnp.float32) + s_ref[1].astype(jnp.float32))
        h = dis * s + b_ref[...]
        o_ref[...] = (dis * jnp.dot(h, w_ref[...],
                                    preferred_element_type=jnp.float32)
                      ).astype(o_ref.dtype)

    return pl.pallas_call(
        body,
        grid=(GRID,),
        in_specs=[pl.BlockSpec((2, RBLK, fin), lambda i: (0, i, 0)),
                  pl.BlockSpec((RBLK, 8), lambda i: (i, 0)),
                  pl.BlockSpec((1, fin), lambda i: (0, 0)),
                  pl.BlockSpec((fin, fout), lambda i: (0, 0))],
        out_specs=pl.BlockSpec((RBLK, fout), lambda i: (i, 0)),
        out_shape=jax.ShapeDtypeStruct((NPAD, fout), jnp.bfloat16),
    )(S, dis8, b, W)


def _tc_final(S, dis8, b3, batch2d, fcW, fcb):
    def body(s_ref, dis_ref, b_ref, batch_ref, fcw_ref, fcb_ref,
             out_ref, sums, cnts):
        i = pl.program_id(0)

        @pl.when(i == 0)
        def _():
            sums[...] = jnp.zeros_like(sums)
            cnts[...] = jnp.zeros_like(cnts)

        dis = dis_ref[:, 0:1]
        s = (s_ref[0].astype(jnp.float32) + s_ref[1].astype(jnp.float32))
        h = dis * s + b_ref[...]
        oh = (batch_ref[...] ==
              lax.broadcasted_iota(jnp.int32, (RBLK, G), 1)
              ).astype(jnp.float32)
        dnum = (((0,), (0,)), ((), ()))
        sums[...] += lax.dot_general(oh, h, dnum,
                                     preferred_element_type=jnp.float32)
        cnts[...] += lax.dot_general(oh, jnp.ones((RBLK, G), jnp.float32),
                                     dnum, preferred_element_type=jnp.float32)

        @pl.when(i == GRID - 1)
        def _():
            pooled = sums[...] / jnp.maximum(cnts[...], 1.0)
            logits = jnp.dot(pooled, fcw_ref[...],
                             preferred_element_type=jnp.float32) + fcb_ref[...]
            m = jnp.max(logits, axis=1, keepdims=True)
            lse = jnp.log(jnp.sum(jnp.exp(logits - m), axis=1,
                                  keepdims=True)) + m
            out_ref[...] = logits - lse

    return pl.pallas_call(
        body,
        grid=(GRID,),
        in_specs=[pl.BlockSpec((2, RBLK, G), lambda i: (0, i, 0)),
                  pl.BlockSpec((RBLK, 8), lambda i: (i, 0)),
                  pl.BlockSpec((1, G), lambda i: (0, 0)),
                  pl.BlockSpec((RBLK, 1), lambda i: (i, 0)),
                  pl.BlockSpec((G, OUT), lambda i: (0, 0)),
                  pl.BlockSpec((1, OUT), lambda i: (0, 0))],
        out_specs=pl.BlockSpec((G, OUT), lambda i: (0, 0)),
        out_shape=jax.ShapeDtypeStruct((G, OUT), jnp.float32),
        scratch_shapes=[pltpu.VMEM((G, G), jnp.float32),
                        pltpu.VMEM((G, G), jnp.float32)],
    )(S, dis8, b3, batch2d, fcW, fcb)


def kernel(x, edge_index, batch, W1, b1, W2, b2, W3, b3, fcW, fcb):
    xp = jnp.concatenate([x, jnp.zeros((NPAD - N, x.shape[1]), jnp.float32)])
    rowflat = jnp.concatenate([edge_index[0], jnp.zeros((EPAD - E,), jnp.int32)])
    colflat = jnp.concatenate([edge_index[1], jnp.full((EPAD - E,), N, jnp.int32)])
    row128 = rowflat.reshape(-1, 128)
    col128 = colflat.reshape(-1, 128)
    batch2d = jnp.concatenate(
        [batch, jnp.full((NPAD - N,), G, jnp.int32)]).reshape(NPAD, 1)
    ones16 = jnp.ones((NPAD, 16), jnp.float32)

    BF = jnp.bfloat16
    F32 = jnp.float32
    degp = _sc_pass(16, False, 128, 4, None, F32, ones16, ones16,
                    row128, col128)
    dis8, v1 = _tc_prep(degp, xp, W1)
    S1 = _sc_pass(160, True, 128, 4, SPLIT128, BF, v1, v1, row128, col128)
    v2 = _tc_combine(S1, dis8, b1.reshape(1, -1), W2, 160, 160)
    S2 = _sc_pass(160, True, 128, 4, SPLIT128, BF, v2, v2, row128, col128)
    v3 = _tc_combine(S2, dis8, b2.reshape(1, -1), W3, 160, 128)
    S3 = _sc_pass(128, True, 128, 4, SPLIT128, BF, v3, v3, row128, col128)
    return _tc_final(S3, dis8, b3.reshape(1, -1), batch2d,
                     fcW, fcb.reshape(1, -1))
